# Initial kernel scaffold; baseline (speedup 1.0000x reference)
import jax
import jax.numpy as jnp
from jax.experimental import pallas as pl


def kernel(x, edge_index, batch, W1, b1, W2, b2, Wl, bl):
    raise NotImplementedError("write your pallas kernel here")


# speedup vs baseline: 22.8442x; 22.8442x over previous
import functools

import jax
import jax.numpy as jnp
from jax import lax
from jax.experimental import pallas as pl
from jax.experimental.pallas import tpu as pltpu
from jax.experimental.pallas import tpu_sc as plsc

N_NODES = 10000
N_EDGES = 320000
D = 128
N_GRAPHS = 64
D_OUT = 40

NPAD = 10240
NW = 32
CHUNK = 128
NCH = 79
EP = NW * NCH * CHUNK
EPW = NCH * CHUNK
ROWS_PT = NPAD // 16

@functools.cache
def _sc_kernels():
    mesh = plsc.VectorSubcoreMesh(core_axis_name="c", subcore_axis_name="s")
    deg_call = functools.partial(
        pl.kernel,
        out_type=jax.ShapeDtypeStruct((2, NPAD), jnp.float32),
        mesh=mesh,
        scratch_types=[
            pltpu.VMEM((NCH, CHUNK), jnp.int32),
            pltpu.VMEM((CHUNK,), jnp.float32),
            pltpu.VMEM_SHARED((NPAD,), jnp.float32),
        ],
    )(_deg_body)
    spmm_call = functools.partial(
        pl.kernel,
        out_type=jax.ShapeDtypeStruct((2, NPAD, D), jnp.float32),
        mesh=mesh,
        scratch_types=[
            pltpu.VMEM((NCH, CHUNK), jnp.int32),
            pltpu.VMEM((NCH, CHUNK), jnp.int32),
            pltpu.VMEM((CHUNK, D), jnp.float32),
            pltpu.VMEM_SHARED((NPAD, D), jnp.float32),
            pltpu.SemaphoreType.DMA,
        ],
    )(_spmm_body)
    return deg_call, spmm_call


def _deg_body(dst_hbm, zeros_hbm, out_hbm, dstv, onesv, dacc):
    cid = lax.axis_index("c")
    sid = lax.axis_index("s")
    wid = sid * 2 + cid
    pltpu.sync_copy(dst_hbm.at[wid], dstv)
    for k in range(CHUNK // 16):
        onesv[pl.ds(16 * k, 16)] = jnp.ones((16,), jnp.float32)

    pltpu.sync_copy(zeros_hbm.at[pl.ds(sid * ROWS_PT, ROWS_PT)],
                    dacc.at[pl.ds(sid * ROWS_PT, ROWS_PT)])

    plsc.subcore_barrier()

    def body(j, carry):
        pltpu.sync_copy(onesv, dacc.at[dstv.at[j]], add=True)
        return carry

    lax.fori_loop(0, NCH, body, 0)
    plsc.subcore_barrier()
    pltpu.sync_copy(dacc.at[pl.ds(sid * ROWS_PT, ROWS_PT)],
                    out_hbm.at[cid, pl.ds(sid * ROWS_PT, ROWS_PT)])


def _spmm_body(hp_hbm, src_hbm, dst_hbm, zeros_hbm, out_hbm,
               srcv, dstv, rows, acc, sem):
    cid = lax.axis_index("c")
    sid = lax.axis_index("s")
    wid = sid * 2 + cid
    pltpu.sync_copy(src_hbm.at[wid], srcv)
    pltpu.sync_copy(dst_hbm.at[wid], dstv)

    @pl.when(cid == 0)
    def _():
        pltpu.sync_copy(hp_hbm.at[pl.ds(sid * ROWS_PT, ROWS_PT)],
                        acc.at[pl.ds(sid * ROWS_PT, ROWS_PT)])

    @pl.when(cid == 1)
    def _():
        pltpu.sync_copy(zeros_hbm.at[pl.ds(sid * ROWS_PT, ROWS_PT)],
                        acc.at[pl.ds(sid * ROWS_PT, ROWS_PT)])

    plsc.subcore_barrier()

    def body(j, carry):
        pltpu.async_copy(hp_hbm.at[srcv.at[j]], rows, sem).wait()
        pltpu.sync_copy(rows, acc.at[dstv.at[j]], add=True)
        return carry

    lax.fori_loop(0, NCH, body, 0)
    plsc.subcore_barrier()
    pltpu.sync_copy(acc.at[pl.ds(sid * ROWS_PT, ROWS_PT)],
                    out_hbm.at[cid, pl.ds(sid * ROWS_PT, ROWS_PT)])


BR = 512
GRID = NPAD // BR


def _dinv_of(deg_ref, i):
    deg = deg_ref[0, :] + deg_ref[1, :] + 1.0
    row = i * BR + lax.broadcasted_iota(jnp.int32, (BR,), 0)
    return jnp.where(row < N_NODES, lax.rsqrt(deg), 0.0)


def _mm_scale_body(deg_ref, x_ref, w_ref, o_ref):
    i = pl.program_id(0)
    dinv = _dinv_of(deg_ref, i)
    h = jax.lax.dot_general(x_ref[...], w_ref[...], (((1,), (0,)), ((), ())),
                            precision=lax.Precision.HIGHEST,
                            preferred_element_type=jnp.float32)
    o_ref[...] = dinv[:, None] * h


def _mm_scale(deg2, xp, W):
    return pl.pallas_call(
        _mm_scale_body,
        grid=(GRID,),
        in_specs=[
            pl.BlockSpec((2, BR), lambda i: (0, i)),
            pl.BlockSpec((BR, D), lambda i: (i, 0)),
            pl.BlockSpec((D, D), lambda i: (0, 0)),
        ],
        out_specs=pl.BlockSpec((BR, D), lambda i: (i, 0)),
        out_shape=jax.ShapeDtypeStruct((NPAD, D), jnp.float32),
    )(deg2, xp, W)


def _mid_body(deg_ref, acc_ref, w_ref, b_ref, o_ref):
    i = pl.program_id(0)
    dinv = _dinv_of(deg_ref, i)
    s = acc_ref[0] + acc_ref[1]
    a = jnp.maximum(dinv[:, None] * s + b_ref[...], 0.0)
    h = jax.lax.dot_general(a, w_ref[...], (((1,), (0,)), ((), ())),
                            precision=lax.Precision.HIGHEST,
                            preferred_element_type=jnp.float32)
    o_ref[...] = dinv[:, None] * h


def _mid(deg2, acc, W, b):
    return pl.pallas_call(
        _mid_body,
        grid=(GRID,),
        in_specs=[
            pl.BlockSpec((2, BR), lambda i: (0, i)),
            pl.BlockSpec((2, BR, D), lambda i: (0, i, 0)),
            pl.BlockSpec((D, D), lambda i: (0, 0)),
            pl.BlockSpec((1, D), lambda i: (0, 0)),
        ],
        out_specs=pl.BlockSpec((BR, D), lambda i: (i, 0)),
        out_shape=jax.ShapeDtypeStruct((NPAD, D), jnp.float32),
    )(deg2, acc, W, b)


def _final_body(deg_ref, acc_ref, b_ref, batch_ref, wl_ref, bl_ref,
                o_ref, sums, cnts):
    i = pl.program_id(0)
    dinv = _dinv_of(deg_ref, i)
    s = acc_ref[0] + acc_ref[1]
    a = jnp.maximum(dinv[:, None] * s + b_ref[...], 0.0)
    g = batch_ref[...]
    onehot_t = (lax.broadcasted_iota(jnp.int32, (N_GRAPHS, BR), 0)
                == g).astype(jnp.float32)
    s_blk = jax.lax.dot_general(onehot_t, a, (((1,), (0,)), ((), ())),
                                precision=lax.Precision.HIGHEST,
                                preferred_element_type=jnp.float32)
    c_blk = jax.lax.dot_general(onehot_t, jnp.ones((BR, D), jnp.float32),
                                (((1,), (0,)), ((), ())),
                                precision=lax.Precision.HIGHEST,
                                preferred_element_type=jnp.float32)

    @pl.when(i == 0)
    def _():
        sums[...] = jnp.zeros_like(sums)
        cnts[...] = jnp.zeros_like(cnts)

    sums[...] += s_blk
    cnts[...] += c_blk

    @pl.when(i == GRID - 1)
    def _():
        p = sums[...] / jnp.maximum(cnts[...], 1.0)
        logits = jax.lax.dot_general(p, wl_ref[...], (((1,), (0,)), ((), ())),
                                     precision=lax.Precision.HIGHEST,
                                     preferred_element_type=jnp.float32)
        logits = logits + bl_ref[...]
        m = jnp.max(logits, axis=1, keepdims=True)
        lse = m + jnp.log(jnp.sum(jnp.exp(logits - m), axis=1, keepdims=True))
        o_ref[...] = logits - lse


def _final(deg2, acc, b, batch2, Wlp, blp):
    return pl.pallas_call(
        _final_body,
        grid=(GRID,),
        in_specs=[
            pl.BlockSpec((2, BR), lambda i: (0, i)),
            pl.BlockSpec((2, BR, D), lambda i: (0, i, 0)),
            pl.BlockSpec((1, D), lambda i: (0, 0)),
            pl.BlockSpec((1, BR), lambda i: (0, i)),
            pl.BlockSpec((D, D), lambda i: (0, 0)),
            pl.BlockSpec((1, D), lambda i: (0, 0)),
        ],
        out_specs=pl.BlockSpec((N_GRAPHS, D), lambda i: (0, 0)),
        out_shape=jax.ShapeDtypeStruct((N_GRAPHS, D), jnp.float32),
        scratch_shapes=[
            pltpu.VMEM((N_GRAPHS, D), jnp.float32),
            pltpu.VMEM((N_GRAPHS, D), jnp.float32),
        ],
    )(deg2, acc, b, batch2, Wlp, blp)


def kernel(x, edge_index, batch, W1, b1, W2, b2, Wl, bl):
    xp = jnp.pad(x, ((0, NPAD - N_NODES), (0, 0)))
    src = edge_index[0].astype(jnp.int32)
    dst = edge_index[1].astype(jnp.int32)
    npad_e = EP - N_EDGES
    fill = N_NODES + (jnp.arange(npad_e, dtype=jnp.int32) % (NPAD - N_NODES))
    src3 = jnp.concatenate([src, fill]).reshape(NW, NCH, CHUNK)
    dst3 = jnp.concatenate([dst, fill]).reshape(NW, NCH, CHUNK)
    batch2 = jnp.pad(batch.astype(jnp.int32), (0, NPAD - N_NODES),
                     constant_values=N_GRAPHS).reshape(1, NPAD)
    b1r = b1.reshape(1, D)
    b2r = b2.reshape(1, D)
    Wlp = jnp.pad(Wl, ((0, 0), (0, D - D_OUT)))
    blp = jnp.pad(bl, (0, D - D_OUT), constant_values=-1e30).reshape(1, D)
    zeros1 = jnp.zeros((NPAD,), jnp.float32)
    zeros2 = jnp.zeros((NPAD, D), jnp.float32)

    deg_call, spmm_call = _sc_kernels()
    deg2 = deg_call(dst3, zeros1)
    hp1 = _mm_scale(deg2, xp, W1)
    acc1 = spmm_call(hp1, src3, dst3, zeros2)
    hp2 = _mid(deg2, acc1, W2, b1r)
    acc2 = spmm_call(hp2, src3, dst3, zeros2)
    out128 = _final(deg2, acc2, b2r, batch2, Wlp, blp)
    return out128[:, :D_OUT]


# speedup vs baseline: 32.5074x; 1.4230x over previous
import functools

import jax
import jax.numpy as jnp
from jax import lax
from jax.experimental import pallas as pl
from jax.experimental.pallas import tpu as pltpu
from jax.experimental.pallas import tpu_sc as plsc

N_NODES = 10000
N_EDGES = 320000
D = 128
N_GRAPHS = 64
D_OUT = 40

NPAD = 10240
NW = 32
CHUNK = 128
NCH = 80
EP = NW * NCH * CHUNK
EPW = NCH * CHUNK
ROWS_PT = NPAD // 16

@functools.cache
def _sc_kernels():
    mesh = plsc.VectorSubcoreMesh(core_axis_name="c", subcore_axis_name="s")
    deg_call = functools.partial(
        pl.kernel,
        out_type=jax.ShapeDtypeStruct((2, NPAD), jnp.float32),
        mesh=mesh,
        scratch_types=[
            pltpu.VMEM((NCH, CHUNK), jnp.int32),
            pltpu.VMEM((CHUNK,), jnp.float32),
            pltpu.VMEM_SHARED((NPAD,), jnp.float32),
        ],
    )(_deg_body)
    spmm_call = functools.partial(
        pl.kernel,
        out_type=jax.ShapeDtypeStruct((2, NPAD, D), jnp.float32),
        mesh=mesh,
        scratch_types=[
            pltpu.VMEM((2, CHUNK), jnp.int32),
            pltpu.VMEM((2, CHUNK), jnp.int32),
            pltpu.VMEM((2, CHUNK), jnp.int32),
            pltpu.VMEM((2, CHUNK), jnp.int32),
            pltpu.VMEM((CHUNK, D), jnp.float32),
            pltpu.VMEM((CHUNK, D), jnp.float32),
            pltpu.VMEM_SHARED((NPAD, D), jnp.float32),
            pltpu.SemaphoreType.DMA,
            pltpu.SemaphoreType.DMA,
            pltpu.SemaphoreType.DMA,
            pltpu.SemaphoreType.DMA,
            pltpu.SemaphoreType.DMA,
            pltpu.SemaphoreType.DMA,
            pltpu.SemaphoreType.DMA,
            pltpu.SemaphoreType.DMA,
        ],
    )(_spmm_body)
    return deg_call, spmm_call


def _deg_body(dst_hbm, zeros_hbm, out_hbm, dstv, onesv, dacc):
    cid = lax.axis_index("c")
    sid = lax.axis_index("s")
    wid = sid * 2 + cid
    pltpu.sync_copy(dst_hbm.at[wid], dstv)
    for k in range(CHUNK // 16):
        onesv[pl.ds(16 * k, 16)] = jnp.ones((16,), jnp.float32)

    pltpu.sync_copy(zeros_hbm.at[pl.ds(sid * ROWS_PT, ROWS_PT)],
                    dacc.at[pl.ds(sid * ROWS_PT, ROWS_PT)])

    plsc.subcore_barrier()

    def body(j, carry):
        pltpu.sync_copy(onesv, dacc.at[dstv.at[j]], add=True)
        return carry

    lax.fori_loop(0, NCH, body, 0)
    plsc.subcore_barrier()
    pltpu.sync_copy(dacc.at[pl.ds(sid * ROWS_PT, ROWS_PT)],
                    out_hbm.at[cid, pl.ds(sid * ROWS_PT, ROWS_PT)])


def _spmm_body(hp_hbm, idx_hbm, zeros_hbm, out_hbm,
               idx0, idx1, idx2, idx3, rows0, rows1, acc,
               si0, si1, si2, si3, sg0, sg1, ss0, ss1):
    cid = lax.axis_index("c")
    sid = lax.axis_index("s")
    wid = sid * 2 + cid
    idxb = (idx0, idx1, idx2, idx3)
    si = (si0, si1, si2, si3)
    rows = (rows0, rows1)
    sg = (sg0, sg1)
    ss = (ss0, ss1)

    @pl.when(cid == 0)
    def _():
        pltpu.sync_copy(hp_hbm.at[pl.ds(sid * ROWS_PT, ROWS_PT)],
                        acc.at[pl.ds(sid * ROWS_PT, ROWS_PT)])

    @pl.when(cid == 1)
    def _():
        pltpu.sync_copy(zeros_hbm.at[pl.ds(sid * ROWS_PT, ROWS_PT)],
                        acc.at[pl.ds(sid * ROWS_PT, ROWS_PT)])

    plsc.subcore_barrier()

    def i_start(c, b):
        pltpu.async_copy(idx_hbm.at[wid, c], idxb[b], si[b])

    def i_wait(b):
        pltpu.make_async_copy(idx_hbm.at[0, 0], idxb[b], si[b]).wait()

    def g_start(c, b, ib):
        pltpu.async_copy(hp_hbm.at[idxb[ib].at[0]], rows[b], sg[b])

    def g_wait(b):
        pltpu.make_async_copy(hp_hbm.at[pl.ds(0, CHUNK)], rows[b],
                              sg[b]).wait()

    def s_start(b, ib):
        pltpu.async_copy(rows[b], acc.at[idxb[ib].at[1]], ss[b], add=True)

    def s_wait(b):
        pltpu.make_async_copy(hp_hbm.at[pl.ds(0, CHUNK)], rows[b],
                              ss[b]).wait()

    i_start(0, 0)
    i_start(1, 1)
    i_start(2, 2)
    i_wait(0)
    g_start(0, 0, 0)

    def body(i, carry):
        c0 = 4 * i
        for b in range(4):
            c = c0 + b
            rb = b % 2

            @pl.when(c - 1 >= 0)
            def _():
                s_wait((b + 1) % 2)

            @pl.when(c + 1 < NCH)
            def _():
                i_wait((b + 1) % 4)
                g_start(c + 1, (b + 1) % 2, (b + 1) % 4)

            @pl.when(c + 3 < NCH)
            def _():
                i_start(c + 3, (b + 3) % 4)

            g_wait(rb)
            s_start(rb, b % 4)
        return carry

    lax.fori_loop(0, NCH // 4, body, 0)
    s_wait((NCH - 1) % 2)
    plsc.subcore_barrier()
    pltpu.sync_copy(acc.at[pl.ds(sid * ROWS_PT, ROWS_PT)],
                    out_hbm.at[cid, pl.ds(sid * ROWS_PT, ROWS_PT)])


BR = 512
GRID = NPAD // BR


def _dinv_of(deg_ref, i):
    deg = deg_ref[0, :] + deg_ref[1, :] + 1.0
    row = i * BR + lax.broadcasted_iota(jnp.int32, (BR,), 0)
    return jnp.where(row < N_NODES, lax.rsqrt(deg), 0.0)


def _mm_scale_body(deg_ref, x_ref, w_ref, o_ref):
    i = pl.program_id(0)
    dinv = _dinv_of(deg_ref, i)
    h = jax.lax.dot_general(x_ref[...], w_ref[...], (((1,), (0,)), ((), ())),
                            precision=lax.Precision.HIGHEST,
                            preferred_element_type=jnp.float32)
    o_ref[...] = dinv[:, None] * h


def _mm_scale(deg2, xp, W):
    return pl.pallas_call(
        _mm_scale_body,
        grid=(GRID,),
        in_specs=[
            pl.BlockSpec((2, BR), lambda i: (0, i)),
            pl.BlockSpec((BR, D), lambda i: (i, 0)),
            pl.BlockSpec((D, D), lambda i: (0, 0)),
        ],
        out_specs=pl.BlockSpec((BR, D), lambda i: (i, 0)),
        out_shape=jax.ShapeDtypeStruct((NPAD, D), jnp.float32),
    )(deg2, xp, W)


def _mid_body(deg_ref, acc_ref, w_ref, b_ref, o_ref):
    i = pl.program_id(0)
    dinv = _dinv_of(deg_ref, i)
    s = acc_ref[0] + acc_ref[1]
    a = jnp.maximum(dinv[:, None] * s + b_ref[...], 0.0)
    h = jax.lax.dot_general(a, w_ref[...], (((1,), (0,)), ((), ())),
                            precision=lax.Precision.HIGHEST,
                            preferred_element_type=jnp.float32)
    o_ref[...] = dinv[:, None] * h


def _mid(deg2, acc, W, b):
    return pl.pallas_call(
        _mid_body,
        grid=(GRID,),
        in_specs=[
            pl.BlockSpec((2, BR), lambda i: (0, i)),
            pl.BlockSpec((2, BR, D), lambda i: (0, i, 0)),
            pl.BlockSpec((D, D), lambda i: (0, 0)),
            pl.BlockSpec((1, D), lambda i: (0, 0)),
        ],
        out_specs=pl.BlockSpec((BR, D), lambda i: (i, 0)),
        out_shape=jax.ShapeDtypeStruct((NPAD, D), jnp.float32),
    )(deg2, acc, W, b)


def _final_body(deg_ref, acc_ref, b_ref, batch_ref, wl_ref, bl_ref,
                o_ref, sums, cnts):
    i = pl.program_id(0)
    dinv = _dinv_of(deg_ref, i)
    s = acc_ref[0] + acc_ref[1]
    a = jnp.maximum(dinv[:, None] * s + b_ref[...], 0.0)
    g = batch_ref[...]
    onehot_t = (lax.broadcasted_iota(jnp.int32, (N_GRAPHS, BR), 0)
                == g).astype(jnp.float32)
    s_blk = jax.lax.dot_general(onehot_t, a, (((1,), (0,)), ((), ())),
                                precision=lax.Precision.HIGHEST,
                                preferred_element_type=jnp.float32)
    c_blk = jax.lax.dot_general(onehot_t, jnp.ones((BR, D), jnp.float32),
                                (((1,), (0,)), ((), ())),
                                precision=lax.Precision.HIGHEST,
                                preferred_element_type=jnp.float32)

    @pl.when(i == 0)
    def _():
        sums[...] = jnp.zeros_like(sums)
        cnts[...] = jnp.zeros_like(cnts)

    sums[...] += s_blk
    cnts[...] += c_blk

    @pl.when(i == GRID - 1)
    def _():
        p = sums[...] / jnp.maximum(cnts[...], 1.0)
        logits = jax.lax.dot_general(p, wl_ref[...], (((1,), (0,)), ((), ())),
                                     precision=lax.Precision.HIGHEST,
                                     preferred_element_type=jnp.float32)
        logits = logits + bl_ref[...]
        m = jnp.max(logits, axis=1, keepdims=True)
        lse = m + jnp.log(jnp.sum(jnp.exp(logits - m), axis=1, keepdims=True))
        o_ref[...] = logits - lse


def _final(deg2, acc, b, batch2, Wlp, blp):
    return pl.pallas_call(
        _final_body,
        grid=(GRID,),
        in_specs=[
            pl.BlockSpec((2, BR), lambda i: (0, i)),
            pl.BlockSpec((2, BR, D), lambda i: (0, i, 0)),
            pl.BlockSpec((1, D), lambda i: (0, 0)),
            pl.BlockSpec((1, BR), lambda i: (0, i)),
            pl.BlockSpec((D, D), lambda i: (0, 0)),
            pl.BlockSpec((1, D), lambda i: (0, 0)),
        ],
        out_specs=pl.BlockSpec((N_GRAPHS, D), lambda i: (0, 0)),
        out_shape=jax.ShapeDtypeStruct((N_GRAPHS, D), jnp.float32),
        scratch_shapes=[
            pltpu.VMEM((N_GRAPHS, D), jnp.float32),
            pltpu.VMEM((N_GRAPHS, D), jnp.float32),
        ],
    )(deg2, acc, b, batch2, Wlp, blp)


def kernel(x, edge_index, batch, W1, b1, W2, b2, Wl, bl):
    xp = jnp.pad(x, ((0, NPAD - N_NODES), (0, 0)))
    src = edge_index[0].astype(jnp.int32)
    dst = edge_index[1].astype(jnp.int32)
    npad_e = EP - N_EDGES
    fill = N_NODES + (jnp.arange(npad_e, dtype=jnp.int32) % (NPAD - N_NODES))
    src3 = jnp.concatenate([src, fill]).reshape(NW, NCH, CHUNK)
    dst3 = jnp.concatenate([dst, fill]).reshape(NW, NCH, CHUNK)
    idx4 = jnp.stack([src3, dst3], axis=2)
    batch2 = jnp.pad(batch.astype(jnp.int32), (0, NPAD - N_NODES),
                     constant_values=N_GRAPHS).reshape(1, NPAD)
    b1r = b1.reshape(1, D)
    b2r = b2.reshape(1, D)
    Wlp = jnp.pad(Wl, ((0, 0), (0, D - D_OUT)))
    blp = jnp.pad(bl, (0, D - D_OUT), constant_values=-1e30).reshape(1, D)
    zeros1 = jnp.zeros((NPAD,), jnp.float32)
    zeros2 = jnp.zeros((NPAD, D), jnp.float32)

    deg_call, spmm_call = _sc_kernels()
    deg2 = deg_call(dst3, zeros1)
    hp1 = _mm_scale(deg2, xp, W1)
    acc1 = spmm_call(hp1, idx4, zeros2)
    hp2 = _mid(deg2, acc1, W2, b1r)
    acc2 = spmm_call(hp2, idx4, zeros2)
    out128 = _final(deg2, acc2, b2r, batch2, Wlp, blp)
    return out128[:, :D_OUT]


# speedup vs baseline: 33.5556x; 1.0322x over previous
import functools

import jax
import jax.numpy as jnp
from jax import lax
from jax.experimental import pallas as pl
from jax.experimental.pallas import tpu as pltpu
from jax.experimental.pallas import tpu_sc as plsc

N_NODES = 10000
N_EDGES = 320000
D = 128
N_GRAPHS = 64
D_OUT = 40

NPAD = 10240
NW = 32
CHUNK = 96
NCH = 108
EP = NW * NCH * CHUNK
EPW = NCH * CHUNK
ROWS_PT = NPAD // 16

@functools.cache
def _sc_kernels():
    mesh = plsc.VectorSubcoreMesh(core_axis_name="c", subcore_axis_name="s")
    deg_call = functools.partial(
        pl.kernel,
        out_type=jax.ShapeDtypeStruct((2, NPAD), jnp.float32),
        mesh=mesh,
        scratch_types=[
            pltpu.VMEM((NCH, CHUNK), jnp.int32),
            pltpu.VMEM((CHUNK,), jnp.float32),
            pltpu.VMEM_SHARED((NPAD,), jnp.float32),
        ],
    )(_deg_body)
    spmm_call = functools.partial(
        pl.kernel,
        out_type=jax.ShapeDtypeStruct((2, NPAD, D), jnp.float32),
        mesh=mesh,
        scratch_types=(
            [pltpu.VMEM((2, CHUNK), jnp.int32)] * 4
            + [pltpu.VMEM((CHUNK, D), jnp.float32)] * 3
            + [pltpu.VMEM_SHARED((NPAD, D), jnp.float32)]
            + [pltpu.SemaphoreType.DMA] * 10
        ),
    )(_spmm_body)
    return deg_call, spmm_call


def _deg_body(dst_hbm, zeros_hbm, out_hbm, dstv, onesv, dacc):
    cid = lax.axis_index("c")
    sid = lax.axis_index("s")
    wid = sid * 2 + cid
    pltpu.sync_copy(dst_hbm.at[wid], dstv)
    for k in range(CHUNK // 16):
        onesv[pl.ds(16 * k, 16)] = jnp.ones((16,), jnp.float32)

    pltpu.sync_copy(zeros_hbm.at[pl.ds(sid * ROWS_PT, ROWS_PT)],
                    dacc.at[pl.ds(sid * ROWS_PT, ROWS_PT)])

    plsc.subcore_barrier()

    def body(j, carry):
        pltpu.sync_copy(onesv, dacc.at[dstv.at[j]], add=True)
        return carry

    lax.fori_loop(0, NCH, body, 0)
    plsc.subcore_barrier()
    pltpu.sync_copy(dacc.at[pl.ds(sid * ROWS_PT, ROWS_PT)],
                    out_hbm.at[cid, pl.ds(sid * ROWS_PT, ROWS_PT)])


def _spmm_body(hp_hbm, idx_hbm, zeros_hbm, out_hbm, *refs):
    cid = lax.axis_index("c")
    sid = lax.axis_index("s")
    wid = sid * 2 + cid
    idxb = refs[0:4]
    rows = refs[4:7]
    acc = refs[7]
    si = refs[8:12]
    sg = refs[12:15]
    ss = refs[15:18]

    @pl.when(cid == 0)
    def _():
        pltpu.sync_copy(hp_hbm.at[pl.ds(sid * ROWS_PT, ROWS_PT)],
                        acc.at[pl.ds(sid * ROWS_PT, ROWS_PT)])

    @pl.when(cid == 1)
    def _():
        pltpu.sync_copy(zeros_hbm.at[pl.ds(sid * ROWS_PT, ROWS_PT)],
                        acc.at[pl.ds(sid * ROWS_PT, ROWS_PT)])

    plsc.subcore_barrier()

    def i_start(c, b):
        pltpu.async_copy(idx_hbm.at[wid, c], idxb[b], si[b])

    def i_wait(b):
        pltpu.make_async_copy(idx_hbm.at[0, 0], idxb[b], si[b]).wait()

    def g_start(c, b, ib):
        pltpu.async_copy(hp_hbm.at[idxb[ib].at[0]], rows[b], sg[b])

    def g_wait(b):
        pltpu.make_async_copy(hp_hbm.at[pl.ds(0, CHUNK)], rows[b],
                              sg[b]).wait()

    def s_start(b, ib):
        pltpu.async_copy(rows[b], acc.at[idxb[ib].at[1]], ss[b], add=True)

    def s_wait(b):
        pltpu.make_async_copy(hp_hbm.at[pl.ds(0, CHUNK)], rows[b],
                              ss[b]).wait()

    i_start(0, 0)
    i_start(1, 1)
    i_wait(0)
    g_start(0, 0, 0)

    def body(i, carry):
        c0 = 12 * i
        for b in range(12):
            c = c0 + b

            @pl.when(c - 2 >= 0)
            def _():
                s_wait((b + 1) % 3)

            @pl.when(c + 1 < NCH)
            def _():
                i_wait((b + 1) % 4)
                g_start(c + 1, (b + 1) % 3, (b + 1) % 4)

            @pl.when(c + 2 < NCH)
            def _():
                i_start(c + 2, (b + 2) % 4)

            g_wait(b % 3)
            s_start(b % 3, b % 4)
        return carry

    lax.fori_loop(0, NCH // 12, body, 0)
    s_wait((NCH - 2) % 3)
    s_wait((NCH - 1) % 3)
    plsc.subcore_barrier()
    pltpu.sync_copy(acc.at[pl.ds(sid * ROWS_PT, ROWS_PT)],
                    out_hbm.at[cid, pl.ds(sid * ROWS_PT, ROWS_PT)])


BR = 512
GRID = NPAD // BR


def _dinv_of(deg_ref, i):
    deg = deg_ref[0, :] + deg_ref[1, :] + 1.0
    row = i * BR + lax.broadcasted_iota(jnp.int32, (BR,), 0)
    return jnp.where(row < N_NODES, lax.rsqrt(deg), 0.0)


def _mm_scale_body(deg_ref, x_ref, w_ref, o_ref):
    i = pl.program_id(0)
    dinv = _dinv_of(deg_ref, i)
    h = jax.lax.dot_general(x_ref[...], w_ref[...], (((1,), (0,)), ((), ())),
                            precision=lax.Precision.HIGHEST,
                            preferred_element_type=jnp.float32)
    o_ref[...] = dinv[:, None] * h


def _mm_scale(deg2, xp, W):
    return pl.pallas_call(
        _mm_scale_body,
        grid=(GRID,),
        in_specs=[
            pl.BlockSpec((2, BR), lambda i: (0, i)),
            pl.BlockSpec((BR, D), lambda i: (i, 0)),
            pl.BlockSpec((D, D), lambda i: (0, 0)),
        ],
        out_specs=pl.BlockSpec((BR, D), lambda i: (i, 0)),
        out_shape=jax.ShapeDtypeStruct((NPAD, D), jnp.float32),
    )(deg2, xp, W)


def _mid_body(deg_ref, acc_ref, w_ref, b_ref, o_ref):
    i = pl.program_id(0)
    dinv = _dinv_of(deg_ref, i)
    s = acc_ref[0] + acc_ref[1]
    a = jnp.maximum(dinv[:, None] * s + b_ref[...], 0.0)
    h = jax.lax.dot_general(a, w_ref[...], (((1,), (0,)), ((), ())),
                            precision=lax.Precision.HIGHEST,
                            preferred_element_type=jnp.float32)
    o_ref[...] = dinv[:, None] * h


def _mid(deg2, acc, W, b):
    return pl.pallas_call(
        _mid_body,
        grid=(GRID,),
        in_specs=[
            pl.BlockSpec((2, BR), lambda i: (0, i)),
            pl.BlockSpec((2, BR, D), lambda i: (0, i, 0)),
            pl.BlockSpec((D, D), lambda i: (0, 0)),
            pl.BlockSpec((1, D), lambda i: (0, 0)),
        ],
        out_specs=pl.BlockSpec((BR, D), lambda i: (i, 0)),
        out_shape=jax.ShapeDtypeStruct((NPAD, D), jnp.float32),
    )(deg2, acc, W, b)


def _final_body(deg_ref, acc_ref, b_ref, batch_ref, wl_ref, bl_ref,
                o_ref, sums, cnts):
    i = pl.program_id(0)
    dinv = _dinv_of(deg_ref, i)
    s = acc_ref[0] + acc_ref[1]
    a = jnp.maximum(dinv[:, None] * s + b_ref[...], 0.0)
    g = batch_ref[...]
    onehot_t = (lax.broadcasted_iota(jnp.int32, (N_GRAPHS, BR), 0)
                == g).astype(jnp.float32)
    s_blk = jax.lax.dot_general(onehot_t, a, (((1,), (0,)), ((), ())),
                                precision=lax.Precision.HIGHEST,
                                preferred_element_type=jnp.float32)
    c_blk = jax.lax.dot_general(onehot_t, jnp.ones((BR, D), jnp.float32),
                                (((1,), (0,)), ((), ())),
                                precision=lax.Precision.HIGHEST,
                                preferred_element_type=jnp.float32)

    @pl.when(i == 0)
    def _():
        sums[...] = jnp.zeros_like(sums)
        cnts[...] = jnp.zeros_like(cnts)

    sums[...] += s_blk
    cnts[...] += c_blk

    @pl.when(i == GRID - 1)
    def _():
        p = sums[...] / jnp.maximum(cnts[...], 1.0)
        logits = jax.lax.dot_general(p, wl_ref[...], (((1,), (0,)), ((), ())),
                                     precision=lax.Precision.HIGHEST,
                                     preferred_element_type=jnp.float32)
        logits = logits + bl_ref[...]
        m = jnp.max(logits, axis=1, keepdims=True)
        lse = m + jnp.log(jnp.sum(jnp.exp(logits - m), axis=1, keepdims=True))
        o_ref[...] = logits - lse


def _final(deg2, acc, b, batch2, Wlp, blp):
    return pl.pallas_call(
        _final_body,
        grid=(GRID,),
        in_specs=[
            pl.BlockSpec((2, BR), lambda i: (0, i)),
            pl.BlockSpec((2, BR, D), lambda i: (0, i, 0)),
            pl.BlockSpec((1, D), lambda i: (0, 0)),
            pl.BlockSpec((1, BR), lambda i: (0, i)),
            pl.BlockSpec((D, D), lambda i: (0, 0)),
            pl.BlockSpec((1, D), lambda i: (0, 0)),
        ],
        out_specs=pl.BlockSpec((N_GRAPHS, D), lambda i: (0, 0)),
        out_shape=jax.ShapeDtypeStruct((N_GRAPHS, D), jnp.float32),
        scratch_shapes=[
            pltpu.VMEM((N_GRAPHS, D), jnp.float32),
            pltpu.VMEM((N_GRAPHS, D), jnp.float32),
        ],
    )(deg2, acc, b, batch2, Wlp, blp)


def kernel(x, edge_index, batch, W1, b1, W2, b2, Wl, bl):
    xp = jnp.pad(x, ((0, NPAD - N_NODES), (0, 0)))
    src = edge_index[0].astype(jnp.int32)
    dst = edge_index[1].astype(jnp.int32)
    npad_e = EP - N_EDGES
    fill = N_NODES + (jnp.arange(npad_e, dtype=jnp.int32) % (NPAD - N_NODES))
    src3 = jnp.concatenate([src, fill]).reshape(NW, NCH, CHUNK)
    dst3 = jnp.concatenate([dst, fill]).reshape(NW, NCH, CHUNK)
    idx4 = jnp.stack([src3, dst3], axis=2)
    batch2 = jnp.pad(batch.astype(jnp.int32), (0, NPAD - N_NODES),
                     constant_values=N_GRAPHS).reshape(1, NPAD)
    b1r = b1.reshape(1, D)
    b2r = b2.reshape(1, D)
    Wlp = jnp.pad(Wl, ((0, 0), (0, D - D_OUT)))
    blp = jnp.pad(bl, (0, D - D_OUT), constant_values=-1e30).reshape(1, D)
    zeros1 = jnp.zeros((NPAD,), jnp.float32)
    zeros2 = jnp.zeros((NPAD, D), jnp.float32)

    deg_call, spmm_call = _sc_kernels()
    deg2 = deg_call(dst3, zeros1)
    hp1 = _mm_scale(deg2, xp, W1)
    acc1 = spmm_call(hp1, idx4, zeros2)
    hp2 = _mid(deg2, acc1, W2, b1r)
    acc2 = spmm_call(hp2, idx4, zeros2)
    out128 = _final(deg2, acc2, b2r, batch2, Wlp, blp)
    return out128[:, :D_OUT]


# speedup vs baseline: 36.1675x; 1.0778x over previous
import functools

import jax
import jax.numpy as jnp
from jax import lax
from jax.experimental import pallas as pl
from jax.experimental.pallas import tpu as pltpu
from jax.experimental.pallas import tpu_sc as plsc

N_NODES = 10000
N_EDGES = 320000
D = 128
N_GRAPHS = 64
D_OUT = 40

NPAD = 10240
NW = 32
CHUNK = 96
NCH = 108
EP = NW * NCH * CHUNK
EPW = NCH * CHUNK
ROWS_PT = NPAD // 16

@functools.cache
def _sc_kernels():
    mesh = plsc.VectorSubcoreMesh(core_axis_name="c", subcore_axis_name="s")
    deg_call = functools.partial(
        pl.kernel,
        out_type=jax.ShapeDtypeStruct((2, NPAD), jnp.float32),
        mesh=mesh,
        scratch_types=[
            pltpu.VMEM((NCH, CHUNK), jnp.int32),
            pltpu.VMEM((CHUNK,), jnp.float32),
            pltpu.VMEM((ROWS_PT,), jnp.float32),
            pltpu.VMEM_SHARED((NPAD,), jnp.float32),
        ],
    )(_deg_body)
    spmm_call = functools.partial(
        pl.kernel,
        out_type=jax.ShapeDtypeStruct((2, NPAD, D), jnp.float32),
        mesh=mesh,
        scratch_types=(
            [pltpu.VMEM((2, CHUNK), jnp.int32)] * 4
            + [pltpu.VMEM((CHUNK, D), jnp.float32)] * 3
            + [pltpu.VMEM_SHARED((NPAD, D), jnp.float32)]
            + [pltpu.SemaphoreType.DMA] * 10
        ),
    )(_spmm_body)
    return deg_call, spmm_call


def _deg_body(eidx_hbm, out_hbm, dstv, onesv, zrow, dacc):
    cid = lax.axis_index("c")
    sid = lax.axis_index("s")
    wid = sid * 2 + cid
    pltpu.sync_copy(eidx_hbm.at[1, wid], dstv)
    for k in range(CHUNK // 16):
        onesv[pl.ds(16 * k, 16)] = jnp.ones((16,), jnp.float32)

    def _zb(j, carry):
        zrow[pl.ds(j * 16, 16)] = jnp.zeros((16,), jnp.float32)
        return carry

    lax.fori_loop(0, ROWS_PT // 16, _zb, 0)
    pltpu.sync_copy(zrow, dacc.at[pl.ds(sid * ROWS_PT, ROWS_PT)])

    plsc.subcore_barrier()

    def body(j, carry):
        pltpu.sync_copy(onesv, dacc.at[dstv.at[j]], add=True)
        return carry

    lax.fori_loop(0, NCH, body, 0)
    plsc.subcore_barrier()
    pltpu.sync_copy(dacc.at[pl.ds(sid * ROWS_PT, ROWS_PT)],
                    out_hbm.at[cid, pl.ds(sid * ROWS_PT, ROWS_PT)])


def _spmm_body(hp_hbm, eidx_hbm, out_hbm, *refs):
    cid = lax.axis_index("c")
    sid = lax.axis_index("s")
    wid = sid * 2 + cid
    idxb = refs[0:4]
    rows = refs[4:7]
    acc = refs[7]
    si = refs[8:12]
    sg = refs[12:15]
    ss = refs[15:18]

    @pl.when(cid == 0)
    def _():
        pltpu.sync_copy(hp_hbm.at[pl.ds(sid * ROWS_PT, ROWS_PT)],
                        acc.at[pl.ds(sid * ROWS_PT, ROWS_PT)])

    @pl.when(cid == 1)
    def _():
        def _zb(j, carry):
            for k in range(D // 16):
                rows[0][j, pl.ds(16 * k, 16)] = jnp.zeros((16,), jnp.float32)
            return carry

        lax.fori_loop(0, CHUNK, _zb, 0)
        base = sid * ROWS_PT
        for r in range(ROWS_PT // CHUNK):
            pltpu.sync_copy(rows[0], acc.at[pl.ds(base + r * CHUNK, CHUNK)])
        rem = ROWS_PT % CHUNK
        if rem:
            pltpu.sync_copy(
                rows[0].at[pl.ds(0, rem)],
                acc.at[pl.ds(base + (ROWS_PT // CHUNK) * CHUNK, rem)])

    plsc.subcore_barrier()

    def i_start(c, b):
        pltpu.async_copy(eidx_hbm.at[0, wid, c], idxb[b].at[0], si[b])
        pltpu.async_copy(eidx_hbm.at[1, wid, c], idxb[b].at[1], si[b])

    def i_wait(b):
        pltpu.make_async_copy(eidx_hbm.at[0, 0, 0], idxb[b].at[0], si[b]).wait()
        pltpu.make_async_copy(eidx_hbm.at[0, 0, 0], idxb[b].at[1], si[b]).wait()

    def g_start(c, b, ib):
        pltpu.async_copy(hp_hbm.at[idxb[ib].at[0]], rows[b], sg[b])

    def g_wait(b):
        pltpu.make_async_copy(hp_hbm.at[pl.ds(0, CHUNK)], rows[b],
                              sg[b]).wait()

    def s_start(b, ib):
        pltpu.async_copy(rows[b], acc.at[idxb[ib].at[1]], ss[b], add=True)

    def s_wait(b):
        pltpu.make_async_copy(hp_hbm.at[pl.ds(0, CHUNK)], rows[b],
                              ss[b]).wait()

    i_start(0, 0)
    i_start(1, 1)
    i_wait(0)
    g_start(0, 0, 0)

    def body(i, carry):
        c0 = 12 * i
        for b in range(12):
            c = c0 + b

            @pl.when(c - 2 >= 0)
            def _():
                s_wait((b + 1) % 3)

            @pl.when(c + 1 < NCH)
            def _():
                i_wait((b + 1) % 4)
                g_start(c + 1, (b + 1) % 3, (b + 1) % 4)

            @pl.when(c + 2 < NCH)
            def _():
                i_start(c + 2, (b + 2) % 4)

            g_wait(b % 3)
            s_start(b % 3, b % 4)
        return carry

    lax.fori_loop(0, NCH // 12, body, 0)
    s_wait((NCH - 2) % 3)
    s_wait((NCH - 1) % 3)
    plsc.subcore_barrier()
    pltpu.sync_copy(acc.at[pl.ds(sid * ROWS_PT, ROWS_PT)],
                    out_hbm.at[cid, pl.ds(sid * ROWS_PT, ROWS_PT)])


BR = 512
GRID = NPAD // BR


def _dinv_of(deg_ref, i):
    deg = deg_ref[0, :] + deg_ref[1, :] + 1.0
    row = i * BR + lax.broadcasted_iota(jnp.int32, (BR,), 0)
    return jnp.where(row < N_NODES, lax.rsqrt(deg), 0.0)


def _mm_scale_body(deg_ref, x_ref, w_ref, o_ref):
    i = pl.program_id(0)
    dinv = _dinv_of(deg_ref, i)
    h = jax.lax.dot_general(x_ref[...], w_ref[...], (((1,), (0,)), ((), ())),
                            preferred_element_type=jnp.float32)
    o_ref[...] = dinv[:, None] * h


def _mm_scale(deg2, xp, W):
    return pl.pallas_call(
        _mm_scale_body,
        grid=(GRID,),
        in_specs=[
            pl.BlockSpec((2, BR), lambda i: (0, i)),
            pl.BlockSpec((BR, D), lambda i: (i, 0)),
            pl.BlockSpec((D, D), lambda i: (0, 0)),
        ],
        out_specs=pl.BlockSpec((BR, D), lambda i: (i, 0)),
        out_shape=jax.ShapeDtypeStruct((NPAD, D), jnp.float32),
    )(deg2, xp, W)


def _mid_body(deg_ref, acc_ref, w_ref, b_ref, o_ref):
    i = pl.program_id(0)
    dinv = _dinv_of(deg_ref, i)
    s = acc_ref[0] + acc_ref[1]
    a = jnp.maximum(dinv[:, None] * s + b_ref[...], 0.0)
    h = jax.lax.dot_general(a, w_ref[...], (((1,), (0,)), ((), ())),
                            preferred_element_type=jnp.float32)
    o_ref[...] = dinv[:, None] * h


def _mid(deg2, acc, W, b):
    return pl.pallas_call(
        _mid_body,
        grid=(GRID,),
        in_specs=[
            pl.BlockSpec((2, BR), lambda i: (0, i)),
            pl.BlockSpec((2, BR, D), lambda i: (0, i, 0)),
            pl.BlockSpec((D, D), lambda i: (0, 0)),
            pl.BlockSpec((1, D), lambda i: (0, 0)),
        ],
        out_specs=pl.BlockSpec((BR, D), lambda i: (i, 0)),
        out_shape=jax.ShapeDtypeStruct((NPAD, D), jnp.float32),
    )(deg2, acc, W, b)


def _final_body(deg_ref, acc_ref, b_ref, batch_ref, wl_ref, bl_ref,
                o_ref, sums, cnts):
    i = pl.program_id(0)
    dinv = _dinv_of(deg_ref, i)
    s = acc_ref[0] + acc_ref[1]
    a = jnp.maximum(dinv[:, None] * s + b_ref[...], 0.0)
    g = batch_ref[...]
    onehot_t = (lax.broadcasted_iota(jnp.int32, (N_GRAPHS, BR), 0)
                == g).astype(jnp.float32)
    s_blk = jax.lax.dot_general(onehot_t, a, (((1,), (0,)), ((), ())),
                                preferred_element_type=jnp.float32)
    c_blk = jax.lax.dot_general(onehot_t, jnp.ones((BR, D), jnp.float32),
                                (((1,), (0,)), ((), ())),
                                preferred_element_type=jnp.float32)

    @pl.when(i == 0)
    def _():
        sums[...] = jnp.zeros_like(sums)
        cnts[...] = jnp.zeros_like(cnts)

    sums[...] += s_blk
    cnts[...] += c_blk

    @pl.when(i == GRID - 1)
    def _():
        p = sums[...] / jnp.maximum(cnts[...], 1.0)
        logits = jax.lax.dot_general(p, wl_ref[...], (((1,), (0,)), ((), ())),
                                     preferred_element_type=jnp.float32)
        logits = logits + bl_ref[...]
        m = jnp.max(logits, axis=1, keepdims=True)
        lse = m + jnp.log(jnp.sum(jnp.exp(logits - m), axis=1, keepdims=True))
        o_ref[...] = logits - lse


def _final(deg2, acc, b, batch2, Wlp, blp):
    return pl.pallas_call(
        _final_body,
        grid=(GRID,),
        in_specs=[
            pl.BlockSpec((2, BR), lambda i: (0, i)),
            pl.BlockSpec((2, BR, D), lambda i: (0, i, 0)),
            pl.BlockSpec((1, D), lambda i: (0, 0)),
            pl.BlockSpec((1, BR), lambda i: (0, i)),
            pl.BlockSpec((D, D), lambda i: (0, 0)),
            pl.BlockSpec((1, D), lambda i: (0, 0)),
        ],
        out_specs=pl.BlockSpec((N_GRAPHS, D), lambda i: (0, 0)),
        out_shape=jax.ShapeDtypeStruct((N_GRAPHS, D), jnp.float32),
        scratch_shapes=[
            pltpu.VMEM((N_GRAPHS, D), jnp.float32),
            pltpu.VMEM((N_GRAPHS, D), jnp.float32),
        ],
    )(deg2, acc, b, batch2, Wlp, blp)


def kernel(x, edge_index, batch, W1, b1, W2, b2, Wl, bl):
    xp = jnp.pad(x, ((0, NPAD - N_NODES), (0, 0)))
    npad_e = EP - N_EDGES
    fill = N_NODES + (jnp.arange(npad_e, dtype=jnp.int32) % (NPAD - N_NODES))
    fill2 = jnp.broadcast_to(fill, (2, npad_e))
    eidx4 = jnp.concatenate([edge_index.astype(jnp.int32), fill2],
                            axis=1).reshape(2, NW, NCH, CHUNK)
    batch2 = jnp.pad(batch.astype(jnp.int32), (0, NPAD - N_NODES),
                     constant_values=N_GRAPHS).reshape(1, NPAD)
    b1r = b1.reshape(1, D)
    b2r = b2.reshape(1, D)
    Wlp = jnp.pad(Wl, ((0, 0), (0, D - D_OUT)))
    blp = jnp.pad(bl, (0, D - D_OUT), constant_values=-1e30).reshape(1, D)
    deg_call, spmm_call = _sc_kernels()
    deg2 = deg_call(eidx4)
    hp1 = _mm_scale(deg2, xp, W1)
    acc1 = spmm_call(hp1, eidx4)
    hp2 = _mid(deg2, acc1, W2, b1r)
    acc2 = spmm_call(hp2, eidx4)
    out128 = _final(deg2, acc2, b2r, batch2, Wlp, blp)
    return out128[:, :D_OUT]


# speedup vs baseline: 37.9038x; 1.0480x over previous
import functools

import jax
import jax.numpy as jnp
from jax import lax
from jax.experimental import pallas as pl
from jax.experimental.pallas import tpu as pltpu
from jax.experimental.pallas import tpu_sc as plsc

N_NODES = 10000
N_EDGES = 320000
D = 128
N_GRAPHS = 64
D_OUT = 40

NPAD = 10240
NW = 32
CHUNK = 80
NCH = 125
EP = NW * NCH * CHUNK
EPW = NCH * CHUNK
ROWS_PT = NPAD // 16

@functools.cache
def _sc_kernels():
    mesh = plsc.VectorSubcoreMesh(core_axis_name="c", subcore_axis_name="s")
    deg_call = functools.partial(
        pl.kernel,
        out_type=jax.ShapeDtypeStruct((2, NPAD), jnp.float32),
        mesh=mesh,
        scratch_types=[
            pltpu.VMEM((NCH // 5, 5, CHUNK), jnp.int32),
            pltpu.VMEM((CHUNK,), jnp.float32),
            pltpu.VMEM((ROWS_PT,), jnp.float32),
            pltpu.VMEM_SHARED((NPAD,), jnp.float32),
        ],
    )(_deg_body)
    spmm_call = functools.partial(
        pl.kernel,
        out_type=jax.ShapeDtypeStruct((2, NPAD, D), jnp.float32),
        mesh=mesh,
        scratch_types=(
            [pltpu.VMEM((5, CHUNK), jnp.int32)] * 3
            + [pltpu.VMEM((5, CHUNK), jnp.int32)] * 3
            + [pltpu.VMEM((CHUNK, D), jnp.float32)] * 3
            + [pltpu.VMEM_SHARED((NPAD, D), jnp.float32)]
            + [pltpu.SemaphoreType.DMA] * 9
        ),
    )(_spmm_body)
    return deg_call, spmm_call


def _deg_body(eidx_hbm, out_hbm, dstv, onesv, zrow, dacc):
    cid = lax.axis_index("c")
    sid = lax.axis_index("s")
    wid = sid * 2 + cid
    pltpu.sync_copy(eidx_hbm.at[1, wid], dstv)
    for k in range(CHUNK // 16):
        onesv[pl.ds(16 * k, 16)] = jnp.ones((16,), jnp.float32)

    def _zb(j, carry):
        zrow[pl.ds(j * 16, 16)] = jnp.zeros((16,), jnp.float32)
        return carry

    lax.fori_loop(0, ROWS_PT // 16, _zb, 0)
    pltpu.sync_copy(zrow, dacc.at[pl.ds(sid * ROWS_PT, ROWS_PT)])

    plsc.subcore_barrier()

    def body(j, carry):
        pltpu.sync_copy(onesv, dacc.at[dstv.at[j // 5, j % 5]], add=True)
        return carry

    lax.fori_loop(0, NCH, body, 0)
    plsc.subcore_barrier()
    pltpu.sync_copy(dacc.at[pl.ds(sid * ROWS_PT, ROWS_PT)],
                    out_hbm.at[cid, pl.ds(sid * ROWS_PT, ROWS_PT)])


def _spmm_body(hp_hbm, eidx_hbm, out_hbm, *refs):
    cid = lax.axis_index("c")
    sid = lax.axis_index("s")
    wid = sid * 2 + cid
    srcb = refs[0:3]
    dstb = refs[3:6]
    rows = refs[6:9]
    acc = refs[9]
    si = refs[10:13]
    sg = refs[13:16]
    ss = refs[16:19]

    @pl.when(cid == 0)
    def _():
        pltpu.sync_copy(hp_hbm.at[pl.ds(sid * ROWS_PT, ROWS_PT)],
                        acc.at[pl.ds(sid * ROWS_PT, ROWS_PT)])

    @pl.when(cid == 1)
    def _():
        def _zb(j, carry):
            for k in range(D // 16):
                rows[0][j, pl.ds(16 * k, 16)] = jnp.zeros((16,), jnp.float32)
            return carry

        lax.fori_loop(0, CHUNK, _zb, 0)
        base = sid * ROWS_PT
        for r in range(ROWS_PT // CHUNK):
            pltpu.sync_copy(rows[0], acc.at[pl.ds(base + r * CHUNK, CHUNK)])
        rem = ROWS_PT % CHUNK
        if rem:
            pltpu.sync_copy(
                rows[0].at[pl.ds(0, rem)],
                acc.at[pl.ds(base + (ROWS_PT // CHUNK) * CHUNK, rem)])

    plsc.subcore_barrier()

    def i_start(s5, b):
        pltpu.async_copy(eidx_hbm.at[0, wid, s5], srcb[b], si[b])
        pltpu.async_copy(eidx_hbm.at[1, wid, s5], dstb[b], si[b])

    def i_wait(b):
        pltpu.make_async_copy(eidx_hbm.at[0, 0, 0], srcb[b], si[b]).wait()
        pltpu.make_async_copy(eidx_hbm.at[0, 0, 0], dstb[b], si[b]).wait()

    def g_start(rb, ib, row):
        pltpu.async_copy(hp_hbm.at[srcb[ib].at[row]], rows[rb], sg[rb])

    def g_wait(b):
        pltpu.make_async_copy(hp_hbm.at[pl.ds(0, CHUNK)], rows[b],
                              sg[b]).wait()

    def s_start(rb, ib, row):
        pltpu.async_copy(rows[rb], acc.at[dstb[ib].at[row]], ss[rb],
                         add=True)

    def s_wait(b):
        pltpu.make_async_copy(hp_hbm.at[pl.ds(0, CHUNK)], rows[b],
                              ss[b]).wait()

    i_start(0, 0)
    i_start(1, 1)
    i_wait(0)
    g_start(0, 0, 0)

    def step(c, b):
        @pl.when(c - 2 >= 0)
        def _():
            s_wait((b + 1) % 3)

        if b % 5 == 2:
            @pl.when(c + 8 < NCH)
            def _():
                i_start((c + 8) // 5, (b // 5 + 2) % 3)

        if b % 5 == 4:
            i_wait((b // 5 + 1) % 3)

        g_start((b + 1) % 3, ((b + 1) // 5) % 3, (b + 1) % 5)
        g_wait(b % 3)
        s_start(b % 3, (b // 5) % 3, b % 5)

    def body(i, carry):
        c0 = 15 * i
        for b in range(15):
            step(c0 + b, b)
        return carry

    lax.fori_loop(0, (NCH - 5) // 15, body, 0)
    for c in range(NCH - 5, NCH):
        s_wait((c - 2) % 3)
        if c + 1 < NCH:
            g_start((c + 1) % 3, 0, (c + 1) % 5)
        g_wait(c % 3)
        s_start(c % 3, 0, c % 5)
    s_wait((NCH - 2) % 3)
    s_wait((NCH - 1) % 3)
    plsc.subcore_barrier()
    pltpu.sync_copy(acc.at[pl.ds(sid * ROWS_PT, ROWS_PT)],
                    out_hbm.at[cid, pl.ds(sid * ROWS_PT, ROWS_PT)])


BR = 1024
GRID = NPAD // BR


def _dinv_of(deg_ref, i):
    deg = deg_ref[0, :] + deg_ref[1, :] + 1.0
    row = i * BR + lax.broadcasted_iota(jnp.int32, (BR,), 0)
    return jnp.where(row < N_NODES, lax.rsqrt(deg), 0.0)


def _mm_scale_body(deg_ref, x_ref, w_ref, o_ref):
    i = pl.program_id(0)
    dinv = _dinv_of(deg_ref, i)
    h = jax.lax.dot_general(x_ref[...], w_ref[...], (((1,), (0,)), ((), ())),
                            preferred_element_type=jnp.float32)
    o_ref[...] = dinv[:, None] * h


def _mm_scale(deg2, xp, W):
    return pl.pallas_call(
        _mm_scale_body,
        grid=(GRID,),
        in_specs=[
            pl.BlockSpec((2, BR), lambda i: (0, i)),
            pl.BlockSpec((BR, D), lambda i: (i, 0)),
            pl.BlockSpec((D, D), lambda i: (0, 0)),
        ],
        out_specs=pl.BlockSpec((BR, D), lambda i: (i, 0)),
        out_shape=jax.ShapeDtypeStruct((NPAD, D), jnp.float32),
    )(deg2, xp, W)


def _mid_body(deg_ref, acc_ref, w_ref, b_ref, o_ref):
    i = pl.program_id(0)
    dinv = _dinv_of(deg_ref, i)
    s = acc_ref[0] + acc_ref[1]
    a = jnp.maximum(dinv[:, None] * s + b_ref[...], 0.0)
    h = jax.lax.dot_general(a, w_ref[...], (((1,), (0,)), ((), ())),
                            preferred_element_type=jnp.float32)
    o_ref[...] = dinv[:, None] * h


def _mid(deg2, acc, W, b):
    return pl.pallas_call(
        _mid_body,
        grid=(GRID,),
        in_specs=[
            pl.BlockSpec((2, BR), lambda i: (0, i)),
            pl.BlockSpec((2, BR, D), lambda i: (0, i, 0)),
            pl.BlockSpec((D, D), lambda i: (0, 0)),
            pl.BlockSpec((1, D), lambda i: (0, 0)),
        ],
        out_specs=pl.BlockSpec((BR, D), lambda i: (i, 0)),
        out_shape=jax.ShapeDtypeStruct((NPAD, D), jnp.float32),
    )(deg2, acc, W, b)


def _final_body(deg_ref, acc_ref, b_ref, batch_ref, wl_ref, bl_ref,
                o_ref, sums, cnts):
    i = pl.program_id(0)
    dinv = _dinv_of(deg_ref, i)
    s = acc_ref[0] + acc_ref[1]
    a = jnp.maximum(dinv[:, None] * s + b_ref[...], 0.0)
    g = batch_ref[...]
    onehot_t = (lax.broadcasted_iota(jnp.int32, (N_GRAPHS, BR), 0)
                == g).astype(jnp.float32)
    s_blk = jax.lax.dot_general(onehot_t, a, (((1,), (0,)), ((), ())),
                                preferred_element_type=jnp.float32)
    c_blk = jax.lax.dot_general(onehot_t, jnp.ones((BR, D), jnp.float32),
                                (((1,), (0,)), ((), ())),
                                preferred_element_type=jnp.float32)

    @pl.when(i == 0)
    def _():
        sums[...] = jnp.zeros_like(sums)
        cnts[...] = jnp.zeros_like(cnts)

    sums[...] += s_blk
    cnts[...] += c_blk

    @pl.when(i == GRID - 1)
    def _():
        p = sums[...] / jnp.maximum(cnts[...], 1.0)
        logits = jax.lax.dot_general(p, wl_ref[...], (((1,), (0,)), ((), ())),
                                     preferred_element_type=jnp.float32)
        logits = logits + bl_ref[...]
        m = jnp.max(logits, axis=1, keepdims=True)
        lse = m + jnp.log(jnp.sum(jnp.exp(logits - m), axis=1, keepdims=True))
        o_ref[...] = logits - lse


def _final(deg2, acc, b, batch2, Wlp, blp):
    return pl.pallas_call(
        _final_body,
        grid=(GRID,),
        in_specs=[
            pl.BlockSpec((2, BR), lambda i: (0, i)),
            pl.BlockSpec((2, BR, D), lambda i: (0, i, 0)),
            pl.BlockSpec((1, D), lambda i: (0, 0)),
            pl.BlockSpec((1, BR), lambda i: (0, i)),
            pl.BlockSpec((D, D), lambda i: (0, 0)),
            pl.BlockSpec((1, D), lambda i: (0, 0)),
        ],
        out_specs=pl.BlockSpec((N_GRAPHS, D), lambda i: (0, 0)),
        out_shape=jax.ShapeDtypeStruct((N_GRAPHS, D), jnp.float32),
        scratch_shapes=[
            pltpu.VMEM((N_GRAPHS, D), jnp.float32),
            pltpu.VMEM((N_GRAPHS, D), jnp.float32),
        ],
    )(deg2, acc, b, batch2, Wlp, blp)


def kernel(x, edge_index, batch, W1, b1, W2, b2, Wl, bl):
    xp = jnp.pad(x, ((0, NPAD - N_NODES), (0, 0)))
    eidx4 = edge_index.astype(jnp.int32).reshape(2, NW, NCH // 5, 5, CHUNK)
    batch2 = jnp.pad(batch.astype(jnp.int32), (0, NPAD - N_NODES),
                     constant_values=N_GRAPHS).reshape(1, NPAD)
    b1r = b1.reshape(1, D)
    b2r = b2.reshape(1, D)
    Wlp = jnp.pad(Wl, ((0, 0), (0, D - D_OUT)))
    blp = jnp.pad(bl, (0, D - D_OUT), constant_values=-1e30).reshape(1, D)
    deg_call, spmm_call = _sc_kernels()
    deg2 = deg_call(eidx4)
    hp1 = _mm_scale(deg2, xp, W1)
    acc1 = spmm_call(hp1, eidx4)
    hp2 = _mid(deg2, acc1, W2, b1r)
    acc2 = spmm_call(hp2, eidx4)
    out128 = _final(deg2, acc2, b2r, batch2, Wlp, blp)
    return out128[:, :D_OUT]
